# baseline (speedup 1.0000x reference)
import functools

import jax
import jax.numpy as jnp
from jax import lax
from jax.experimental import pallas as pl
from jax.experimental.pallas import tpu as pltpu
from jax.experimental.pallas import tpu_sc as plsc

N_NODES = 100000
TOTAL_DIM = 128
SIZES = (16, 64, 320, 400, 400, 400, 400, 400, 400, 240, 112, 48)
OFFS = (0, 16, 80, 400, 800, 1200, 1600, 2000, 2400, 2800, 3040, 3152)
MAXC = 400
NSLOTS = len(SIZES)
SPAN = 3200
LAST_SLOTS = 4
NUM_WORKERS_FULL = 31
NIDX = 3

_mesh = plsc.VectorSubcoreMesh(core_axis_name="c", subcore_axis_name="s")


@functools.partial(
    pl.kernel,
    mesh=_mesh,
    out_type=jax.ShapeDtypeStruct((N_NODES, TOTAL_DIM), jnp.float32),
    scratch_types=[pltpu.VMEM((MAXC,), jnp.int32) for _ in range(NIDX)]
    + [pltpu.VMEM((MAXC, TOTAL_DIM), jnp.float32) for _ in range(2)]
    + [pltpu.SemaphoreType.DMA for _ in range(NIDX + 2)],
)
def _gather_kernel(idx_hbm, table_hbm, out_hbm, *scratch):
    ibufs = scratch[:NIDX]
    rows = scratch[NIDX : NIDX + 2]
    isems = scratch[NIDX + 2 : 2 * NIDX + 2]
    gsems = scratch[2 * NIDX + 2 :]
    wid = lax.axis_index("s") * 2 + lax.axis_index("c")
    base = wid * SPAN
    full = wid < NUM_WORKERS_FULL

    def idesc(j):
        b = j % NIDX
        return pltpu.make_async_copy(
            idx_hbm.at[pl.ds(base + OFFS[j], SIZES[j])],
            ibufs[b].at[pl.ds(0, SIZES[j])],
            isems[b],
        )

    def gdesc(j):
        b = j % 2
        return pltpu.make_async_copy(
            table_hbm.at[ibufs[j % NIDX].at[pl.ds(0, SIZES[j])]],
            rows[b].at[pl.ds(0, SIZES[j])],
            gsems[b],
        )

    def wout(j):
        pltpu.sync_copy(
            rows[j % 2].at[pl.ds(0, SIZES[j])],
            out_hbm.at[pl.ds(base + OFFS[j], SIZES[j])],
        )

    def guarded(j, fn):
        if j < LAST_SLOTS:
            fn()
        else:

            @pl.when(full)
            def _():
                fn()

    idesc(0).start()
    idesc(1).start()
    idesc(0).wait()
    gdesc(0).start()
    for j in range(NSLOTS):
        if j + 2 < NSLOTS:
            guarded(j + 2, lambda j=j: idesc(j + 2).start())
        if j + 1 < NSLOTS:
            guarded(
                j + 1,
                lambda j=j: (idesc(j + 1).wait(), gdesc(j + 1).start()),
            )
        guarded(j, lambda j=j: (gdesc(j).wait(), wout(j)))


def kernel(atomic_numbers, embedding):
    idx = atomic_numbers.astype(jnp.int32)
    return _gather_kernel(idx, embedding)


# speedup vs baseline: 1.0076x; 1.0076x over previous
import functools

import jax
import jax.numpy as jnp
from jax import lax
from jax.experimental import pallas as pl
from jax.experimental.pallas import tpu as pltpu
from jax.experimental.pallas import tpu_sc as plsc

N_NODES = 100000
TOTAL_DIM = 128
SIZES = (48, 352, 400, 400, 400, 400, 400, 400, 320, 80)
OFFS = (0, 48, 400, 800, 1200, 1600, 2000, 2400, 2800, 3120)
MAXC = 400
NSLOTS = len(SIZES)
SPAN = 3200
LAST_SLOTS = 3
NUM_WORKERS_FULL = 31
NIDX = 3

_mesh = plsc.VectorSubcoreMesh(core_axis_name="c", subcore_axis_name="s")


@functools.partial(
    pl.kernel,
    mesh=_mesh,
    out_type=jax.ShapeDtypeStruct((N_NODES, TOTAL_DIM), jnp.float32),
    scratch_types=[pltpu.VMEM((MAXC,), jnp.int32) for _ in range(NIDX)]
    + [pltpu.VMEM((MAXC, TOTAL_DIM), jnp.float32) for _ in range(2)]
    + [pltpu.SemaphoreType.DMA for _ in range(NIDX + 2)],
)
def _gather_kernel(idx_hbm, table_hbm, out_hbm, *scratch):
    ibufs = scratch[:NIDX]
    rows = scratch[NIDX : NIDX + 2]
    isems = scratch[NIDX + 2 : 2 * NIDX + 2]
    gsems = scratch[2 * NIDX + 2 :]
    wid = lax.axis_index("s") * 2 + lax.axis_index("c")
    base = wid * SPAN
    full = wid < NUM_WORKERS_FULL

    def idesc(j):
        b = j % NIDX
        return pltpu.make_async_copy(
            idx_hbm.at[pl.ds(base + OFFS[j], SIZES[j])],
            ibufs[b].at[pl.ds(0, SIZES[j])],
            isems[b],
        )

    def gdesc(j):
        b = j % 2
        return pltpu.make_async_copy(
            table_hbm.at[ibufs[j % NIDX].at[pl.ds(0, SIZES[j])]],
            rows[b].at[pl.ds(0, SIZES[j])],
            gsems[b],
        )

    def wout(j):
        pltpu.sync_copy(
            rows[j % 2].at[pl.ds(0, SIZES[j])],
            out_hbm.at[pl.ds(base + OFFS[j], SIZES[j])],
        )

    def guarded(j, fn):
        if j < LAST_SLOTS:
            fn()
        else:

            @pl.when(full)
            def _():
                fn()

    idesc(0).start()
    idesc(1).start()
    idesc(0).wait()
    gdesc(0).start()
    for j in range(NSLOTS):
        if j + 2 < NSLOTS:
            guarded(j + 2, lambda j=j: idesc(j + 2).start())
        if j + 1 < NSLOTS:
            guarded(
                j + 1,
                lambda j=j: (idesc(j + 1).wait(), gdesc(j + 1).start()),
            )
        guarded(j, lambda j=j: (gdesc(j).wait(), wout(j)))


def kernel(atomic_numbers, embedding):
    idx = atomic_numbers.astype(jnp.int32)
    return _gather_kernel(idx, embedding)


# speedup vs baseline: 1.0109x; 1.0033x over previous
import functools

import jax
import jax.numpy as jnp
from jax import lax
from jax.experimental import pallas as pl
from jax.experimental.pallas import tpu as pltpu
from jax.experimental.pallas import tpu_sc as plsc

N_NODES = 100000
TOTAL_DIM = 128
SIZES = (80, 320, 400, 400, 400, 400, 400, 400, 400)
OFFS = (0, 80, 400, 800, 1200, 1600, 2000, 2400, 2800)
MAXC = 400
NSLOTS = len(SIZES)
SPAN = 3200
LAST_SLOTS = 3
NUM_WORKERS_FULL = 31
NIDX = 3

_mesh = plsc.VectorSubcoreMesh(core_axis_name="c", subcore_axis_name="s")


@functools.partial(
    pl.kernel,
    mesh=_mesh,
    out_type=jax.ShapeDtypeStruct((N_NODES, TOTAL_DIM), jnp.float32),
    scratch_types=[pltpu.VMEM((MAXC,), jnp.int32) for _ in range(NIDX)]
    + [pltpu.VMEM((MAXC, TOTAL_DIM), jnp.float32) for _ in range(2)]
    + [pltpu.SemaphoreType.DMA for _ in range(NIDX + 2)],
)
def _gather_kernel(idx_hbm, table_hbm, out_hbm, *scratch):
    ibufs = scratch[:NIDX]
    rows = scratch[NIDX : NIDX + 2]
    isems = scratch[NIDX + 2 : 2 * NIDX + 2]
    gsems = scratch[2 * NIDX + 2 :]
    wid = lax.axis_index("s") * 2 + lax.axis_index("c")
    base = wid * SPAN
    full = wid < NUM_WORKERS_FULL

    def idesc(j):
        b = j % NIDX
        return pltpu.make_async_copy(
            idx_hbm.at[pl.ds(base + OFFS[j], SIZES[j])],
            ibufs[b].at[pl.ds(0, SIZES[j])],
            isems[b],
        )

    def gdesc(j):
        b = j % 2
        return pltpu.make_async_copy(
            table_hbm.at[ibufs[j % NIDX].at[pl.ds(0, SIZES[j])]],
            rows[b].at[pl.ds(0, SIZES[j])],
            gsems[b],
        )

    def wout(j):
        pltpu.sync_copy(
            rows[j % 2].at[pl.ds(0, SIZES[j])],
            out_hbm.at[pl.ds(base + OFFS[j], SIZES[j])],
        )

    def guarded(j, fn):
        if j < LAST_SLOTS:
            fn()
        else:

            @pl.when(full)
            def _():
                fn()

    idesc(0).start()
    idesc(1).start()
    idesc(0).wait()
    gdesc(0).start()
    for j in range(NSLOTS):
        if j + 2 < NSLOTS:
            guarded(j + 2, lambda j=j: idesc(j + 2).start())
        if j + 1 < NSLOTS:
            guarded(
                j + 1,
                lambda j=j: (idesc(j + 1).wait(), gdesc(j + 1).start()),
            )
        guarded(j, lambda j=j: (gdesc(j).wait(), wout(j)))


def kernel(atomic_numbers, embedding):
    idx = atomic_numbers.astype(jnp.int32)
    return _gather_kernel(idx, embedding)


# speedup vs baseline: 1.0124x; 1.0014x over previous
import functools

import jax
import jax.numpy as jnp
from jax import lax
from jax.experimental import pallas as pl
from jax.experimental.pallas import tpu as pltpu
from jax.experimental.pallas import tpu_sc as plsc

N_NODES = 100000
TOTAL_DIM = 128
SIZES = (80, 320, 400, 480, 480, 480, 480, 480)
OFFS = (0, 80, 400, 800, 1280, 1760, 2240, 2720)
MAXC = 480
NSLOTS = len(SIZES)
SPAN = 3200
LAST_SLOTS = 3
NUM_WORKERS_FULL = 31
NIDX = 3

_mesh = plsc.VectorSubcoreMesh(core_axis_name="c", subcore_axis_name="s")


@functools.partial(
    pl.kernel,
    mesh=_mesh,
    out_type=jax.ShapeDtypeStruct((N_NODES, TOTAL_DIM), jnp.float32),
    scratch_types=[pltpu.VMEM((MAXC,), jnp.int32) for _ in range(NIDX)]
    + [pltpu.VMEM((MAXC, TOTAL_DIM), jnp.float32) for _ in range(2)]
    + [pltpu.SemaphoreType.DMA for _ in range(NIDX + 2)],
)
def _gather_kernel(idx_hbm, table_hbm, out_hbm, *scratch):
    ibufs = scratch[:NIDX]
    rows = scratch[NIDX : NIDX + 2]
    isems = scratch[NIDX + 2 : 2 * NIDX + 2]
    gsems = scratch[2 * NIDX + 2 :]
    wid = lax.axis_index("s") * 2 + lax.axis_index("c")
    base = wid * SPAN
    full = wid < NUM_WORKERS_FULL

    def idesc(j):
        b = j % NIDX
        return pltpu.make_async_copy(
            idx_hbm.at[pl.ds(base + OFFS[j], SIZES[j])],
            ibufs[b].at[pl.ds(0, SIZES[j])],
            isems[b],
        )

    def gdesc(j):
        b = j % 2
        return pltpu.make_async_copy(
            table_hbm.at[ibufs[j % NIDX].at[pl.ds(0, SIZES[j])]],
            rows[b].at[pl.ds(0, SIZES[j])],
            gsems[b],
        )

    def wout(j):
        pltpu.sync_copy(
            rows[j % 2].at[pl.ds(0, SIZES[j])],
            out_hbm.at[pl.ds(base + OFFS[j], SIZES[j])],
        )

    def guarded(j, fn):
        if j < LAST_SLOTS:
            fn()
        else:

            @pl.when(full)
            def _():
                fn()

    idesc(0).start()
    idesc(1).start()
    idesc(0).wait()
    gdesc(0).start()
    for j in range(NSLOTS):
        if j + 2 < NSLOTS:
            guarded(j + 2, lambda j=j: idesc(j + 2).start())
        if j + 1 < NSLOTS:
            guarded(
                j + 1,
                lambda j=j: (idesc(j + 1).wait(), gdesc(j + 1).start()),
            )
        guarded(j, lambda j=j: (gdesc(j).wait(), wout(j)))


def kernel(atomic_numbers, embedding):
    idx = atomic_numbers.astype(jnp.int32)
    return _gather_kernel(idx, embedding)


# speedup vs baseline: 1.0142x; 1.0018x over previous
import functools

import jax
import jax.numpy as jnp
from jax import lax
from jax.experimental import pallas as pl
from jax.experimental.pallas import tpu as pltpu
from jax.experimental.pallas import tpu_sc as plsc

N_NODES = 100000
TOTAL_DIM = 128
SIZES = (80, 320, 400, 400, 400, 400, 400, 400, 320, 80)
OFFS = (0, 80, 400, 800, 1200, 1600, 2000, 2400, 2800, 3120)
MAXC = 400
NSLOTS = len(SIZES)
SPAN = 3200
LAST_SLOTS = 3
NUM_WORKERS_FULL = 31
NIDX = 3

_mesh = plsc.VectorSubcoreMesh(core_axis_name="c", subcore_axis_name="s")


@functools.partial(
    pl.kernel,
    mesh=_mesh,
    out_type=jax.ShapeDtypeStruct((N_NODES, TOTAL_DIM), jnp.float32),
    scratch_types=[pltpu.VMEM((MAXC,), jnp.int32) for _ in range(NIDX)]
    + [pltpu.VMEM((MAXC, TOTAL_DIM), jnp.float32) for _ in range(2)]
    + [pltpu.SemaphoreType.DMA for _ in range(NIDX + 2)],
)
def _gather_kernel(idx_hbm, table_hbm, out_hbm, *scratch):
    ibufs = scratch[:NIDX]
    rows = scratch[NIDX : NIDX + 2]
    isems = scratch[NIDX + 2 : 2 * NIDX + 2]
    gsems = scratch[2 * NIDX + 2 :]
    wid = lax.axis_index("s") * 2 + lax.axis_index("c")
    base = wid * SPAN
    full = wid < NUM_WORKERS_FULL

    def idesc(j):
        b = j % NIDX
        return pltpu.make_async_copy(
            idx_hbm.at[pl.ds(base + OFFS[j], SIZES[j])],
            ibufs[b].at[pl.ds(0, SIZES[j])],
            isems[b],
        )

    def gdesc(j):
        b = j % 2
        return pltpu.make_async_copy(
            table_hbm.at[ibufs[j % NIDX].at[pl.ds(0, SIZES[j])]],
            rows[b].at[pl.ds(0, SIZES[j])],
            gsems[b],
        )

    def wout(j):
        pltpu.sync_copy(
            rows[j % 2].at[pl.ds(0, SIZES[j])],
            out_hbm.at[pl.ds(base + OFFS[j], SIZES[j])],
        )

    def guarded(j, fn):
        if j < LAST_SLOTS:
            fn()
        else:

            @pl.when(full)
            def _():
                fn()

    idesc(0).start()
    idesc(1).start()
    idesc(0).wait()
    gdesc(0).start()
    for j in range(NSLOTS):
        if j + 2 < NSLOTS:
            guarded(j + 2, lambda j=j: idesc(j + 2).start())
        if j + 1 < NSLOTS:
            guarded(
                j + 1,
                lambda j=j: (idesc(j + 1).wait(), gdesc(j + 1).start()),
            )
        guarded(j, lambda j=j: (gdesc(j).wait(), wout(j)))


def kernel(atomic_numbers, embedding):
    idx = atomic_numbers.astype(jnp.int32)
    return _gather_kernel(idx, embedding)
